# Initial kernel scaffold; baseline (speedup 1.0000x reference)
import jax
import jax.numpy as jnp
from jax.experimental import pallas as pl


def kernel(x, edge_index, edge_attr, params):
    raise NotImplementedError("write your pallas kernel here")


# speedup vs baseline: 1.8528x; 1.8528x over previous
import functools

import jax
import jax.numpy as jnp
from jax import lax
from jax.experimental import pallas as pl
from jax.experimental.pallas import tpu as pltpu
from jax.experimental.pallas import tpu_sc as plsc

N = 10000
NE = 160000
D = 128
NP = 10240
TRASH = N
K_POOL = (N + 1) // 2
NC, NS = 2, 16
NW = NC * NS
CHUNK = 128
E_PAD = 163840
EPT = E_PAD // NW
EPT1 = E_PAD // NS
BLK = 1024
GRID = NP // BLK

_mesh_cache = {}


def _vmesh():
    if "m" not in _mesh_cache:
        _mesh_cache["m"] = plsc.VectorSubcoreMesh(
            core_axis_name="c", subcore_axis_name="s",
            num_cores=NC, num_subcores=NS)
    return _mesh_cache["m"]


def _f32(*shape):
    return jax.ShapeDtypeStruct(shape, jnp.float32)


def _i32(*shape):
    return jax.ShapeDtypeStruct(shape, jnp.int32)


def _sc_edge_pass_body(table, rows_h, cols_h, zeros_h, out, acc, ridx, cidx,
                       gbuf, sem):
    c = lax.axis_index("c")
    s = lax.axis_index("s")
    wid = s * NC + c
    rpt = NP // NS
    base = s * rpt
    pltpu.sync_copy(zeros_h.at[pl.ds(base, rpt)], acc.at[pl.ds(base, rpt)])
    plsc.subcore_barrier()

    nchunks = EPT // CHUNK

    def chunk(j, carry):
        pltpu.sync_copy(rows_h.at[wid, pl.ds(j * CHUNK, CHUNK)], ridx)
        pltpu.sync_copy(cols_h.at[wid, pl.ds(j * CHUNK, CHUNK)], cidx)
        pltpu.async_copy(table.at[ridx], gbuf, sem).wait()
        pltpu.sync_copy(gbuf, acc.at[cidx], add=True)
        return carry

    lax.fori_loop(0, nchunks, chunk, 0)
    plsc.subcore_barrier()
    pltpu.sync_copy(acc.at[pl.ds(base, rpt)], out.at[c, pl.ds(base, rpt)])


def _sc_edge_pass(table, rows, cols, zeros_nd):
    if "edge" not in _mesh_cache:
        _mesh_cache["edge"] = pl.kernel(
            _sc_edge_pass_body,
            out_type=_f32(NC, NP, D),
            mesh=_vmesh(),
            scratch_types=[
                pltpu.VMEM_SHARED((NP, D), jnp.float32),
                pltpu.VMEM((CHUNK,), jnp.int32),
                pltpu.VMEM((CHUNK,), jnp.int32),
                pltpu.VMEM((CHUNK, D), jnp.float32),
                pltpu.SemaphoreType.DMA,
            ],
        )
    return _mesh_cache["edge"](table, rows, cols, zeros_nd)


_ROWSPEC = pl.BlockSpec((BLK, D), lambda i: (i, 0))
_S16SPEC = pl.BlockSpec((BLK, 16), lambda i: (i, 0))
_P16SPEC = pl.BlockSpec((NC, BLK, 16), lambda i: (0, i, 0))
_PARTSPEC = pl.BlockSpec((NC, BLK, D), lambda i: (0, i, 0))
_WSPEC = pl.BlockSpec((D, D), lambda i: (0, 0))
_BSPEC = pl.BlockSpec((1, D), lambda i: (0, 0))


def _k0_body(x_ref, ap_ref, bp_ref, w0_ref, hd_ref, dis_ref, d2_ref):
    d2 = ap_ref[0, :, 0:1] + ap_ref[1, :, 0:1]
    deg = bp_ref[0, :, 0:1] + bp_ref[1, :, 0:1] + 1.0
    dis = lax.rsqrt(deg)
    colmask = lax.broadcasted_iota(jnp.int32, (BLK, 16), 1) == 0
    dis_ref[...] = jnp.where(colmask, dis, 0.0)
    d2_ref[...] = jnp.where(colmask, d2, 0.0)
    hd_ref[...] = dis * jnp.dot(x_ref[...], w0_ref[...],
                                preferred_element_type=jnp.float32)


_k0 = pl.pallas_call(
    _k0_body,
    grid=(GRID,),
    in_specs=[_ROWSPEC, _PARTSPEC, _PARTSPEC, _WSPEC],
    out_specs=[_ROWSPEC, _S16SPEC, _S16SPEC],
    out_shape=[_f32(NP, D), _f32(NP, 16), _f32(NP, 16)],
)


def _k1_body(sp_ref, hd_ref, dis_ref, b0_ref, pw_ref, x1_ref, sc_ref, key_ref):
    i = pl.program_id(0)
    dis = dis_ref[:, 0:1]
    agg = sp_ref[0, :, :] + sp_ref[1, :, :] + hd_ref[...]
    x1 = jnp.maximum(dis * agg + b0_ref[...], 0.0)
    x1_ref[...] = x1
    pw = pw_ref[...]
    nrm = jnp.sqrt(jnp.sum(pw * pw))
    s = jnp.tanh(jnp.sum(x1 * pw, axis=1, keepdims=True) / nrm)
    colmask = lax.broadcasted_iota(jnp.int32, (BLK, 16), 1) == 0
    sc_ref[...] = jnp.where(colmask, s, 0.0)
    b = lax.bitcast_convert_type(s, jnp.int32)
    key = jnp.where(b >= 0, b + jnp.int32(0x40000000),
                    jnp.int32(0x40000000) - (b & jnp.int32(0x7FFFFFFF)))
    node = i * BLK + lax.broadcasted_iota(jnp.int32, (BLK, 16), 0)
    valid = (node < N) & colmask
    key_ref[...] = jnp.where(valid, key, jnp.int32(0))


_k1 = pl.pallas_call(
    _k1_body,
    grid=(GRID,),
    in_specs=[_PARTSPEC, _ROWSPEC, _S16SPEC, _BSPEC, _BSPEC],
    out_specs=[_ROWSPEC, _S16SPEC, _S16SPEC],
    out_shape=[_f32(NP, D), _f32(NP, 16), _i32(NP, 16)],
)


def _k2_body(key_ref, m_ref):
    keys = key_ref[...]
    idx = lax.broadcasted_iota(jnp.int32, (NP, 16), 0)
    col0 = lax.broadcasted_iota(jnp.int32, (NP, 16), 1) == 0
    kk = jnp.int32(K_POOL)

    def bit_step(i, t):
        cand = t | (jnp.int32(1) << (jnp.int32(30) - i))
        cnt = jnp.sum((keys >= cand).astype(jnp.int32))
        return jnp.where(cnt >= kk, cand, t)

    thr = lax.fori_loop(0, 31, bit_step, jnp.int32(0))
    cgt = jnp.sum((keys > thr).astype(jnp.int32))
    r = kk - cgt
    eq = (keys == thr) & col0

    def idx_step(i, a):
        cand = a + (jnp.int32(1) << (jnp.int32(13) - i))
        cnt = jnp.sum((eq & (idx <= cand - 1)).astype(jnp.int32))
        return jnp.where(cnt < r, cand, a)

    a = lax.fori_loop(0, 14, idx_step, jnp.int32(0))
    sel = (keys > thr) | (eq & (idx <= a))
    msel = jnp.max(sel.astype(jnp.float32), axis=1, keepdims=True)
    m_ref[...] = jnp.broadcast_to(msel, (NP, D))


_k2 = pl.pallas_call(
    _k2_body,
    grid=(1,),
    in_specs=[pl.BlockSpec((NP, 16), lambda i: (0, 0))],
    out_specs=pl.BlockSpec((NP, D), lambda i: (0, 0)),
    out_shape=_f32(NP, D),
)


def _k3_body(x1_ref, sc_ref, m_ref, t1_ref, t2p_ref, d2_ref, w1_ref,
             hn_ref, dist_ref):
    m = m_ref[:, 0:1]
    t2m = t2p_ref[0, :, 0:1] + t2p_ref[1, :, 0:1]
    dega = t2m + 2.0 * t1_ref[:, 0:1] - d2_ref[:, 0:1] * m
    dist = m * lax.rsqrt(1.0 + dega)
    colmask = lax.broadcasted_iota(jnp.int32, (BLK, 16), 1) == 0
    dist_ref[...] = jnp.where(colmask, dist, 0.0)
    xt = (m * sc_ref[:, 0:1]) * x1_ref[...]
    hn_ref[...] = dist * jnp.dot(xt, w1_ref[...],
                                 preferred_element_type=jnp.float32)


_k3 = pl.pallas_call(
    _k3_body,
    grid=(GRID,),
    in_specs=[_ROWSPEC, _S16SPEC, _ROWSPEC, _ROWSPEC, _PARTSPEC, _S16SPEC,
              _WSPEC],
    out_specs=[_ROWSPEC, _S16SPEC],
    out_shape=[_f32(NP, D), _f32(NP, 16)],
)


def _k3b_body(tp_ref, t1_ref):
    t1_ref[...] = tp_ref[0, :, :] + tp_ref[1, :, :]


_k3b = pl.pallas_call(
    _k3b_body,
    grid=(GRID,),
    in_specs=[_PARTSPEC],
    out_specs=_ROWSPEC,
    out_shape=_f32(NP, D),
)


def _k4_body(t2p_ref, t1_ref, hn_ref, dist_ref, m_ref, d2_ref, x1_ref,
             dis_ref, w2_ref, b1_ref, hd2_ref):
    hn = hn_ref[...]
    y = t2p_ref[0, :, :] + t2p_ref[1, :, :] + 2.0 * t1_ref[...] \
        - d2_ref[:, 0:1] * hn
    x2 = m_ref[:, 0:1] * jnp.maximum(dist_ref[:, 0:1] * (y + hn) + b1_ref[...],
                                     0.0)
    z = x1_ref[...] + x2
    hd2_ref[...] = dis_ref[:, 0:1] * jnp.dot(z, w2_ref[...],
                                             preferred_element_type=jnp.float32)


_k4 = pl.pallas_call(
    _k4_body,
    grid=(GRID,),
    in_specs=[_PARTSPEC, _ROWSPEC, _ROWSPEC, _S16SPEC, _ROWSPEC, _S16SPEC,
              _ROWSPEC, _S16SPEC, _WSPEC, _BSPEC],
    out_specs=_ROWSPEC,
    out_shape=_f32(NP, D),
)


def _k5_body(sp_ref, hd2_ref, dis_ref, b2_ref, w0n_ref, hdn_ref):
    dis = dis_ref[:, 0:1]
    o = dis * (sp_ref[0, :, :] + sp_ref[1, :, :] + hd2_ref[...]) + b2_ref[...]
    xn = jnp.where(o > 0, o, jnp.exp(o) - 1.0)
    hdn_ref[...] = dis * jnp.dot(xn, w0n_ref[...],
                                 preferred_element_type=jnp.float32)


_k5 = pl.pallas_call(
    _k5_body,
    grid=(GRID,),
    in_specs=[_PARTSPEC, _ROWSPEC, _S16SPEC, _BSPEC, _WSPEC],
    out_specs=_ROWSPEC,
    out_shape=_f32(NP, D),
)


def _k5f_body(sp_ref, hd2_ref, dis_ref, b2_ref, out_ref):
    dis = dis_ref[:, 0:1]
    o = dis * (sp_ref[0, :, :] + sp_ref[1, :, :] + hd2_ref[...]) + b2_ref[...]
    out_ref[...] = jnp.where(o > 0, o, float(D) * (jnp.exp(o) - 1.0))


_k5f = pl.pallas_call(
    _k5f_body,
    grid=(GRID,),
    in_specs=[_PARTSPEC, _ROWSPEC, _S16SPEC, _BSPEC],
    out_specs=_ROWSPEC,
    out_shape=_f32(NP, D),
)


def kernel(x, edge_index, edge_attr, params):
    row = edge_index[0].astype(jnp.int32)
    col = edge_index[1].astype(jnp.int32)
    nonself = row != col
    colb = jnp.where(nonself, col, TRASH)

    pad = E_PAD - NE
    rowp = jnp.concatenate([row, jnp.zeros((pad,), jnp.int32)])
    colp = jnp.concatenate([col, jnp.full((pad,), TRASH, jnp.int32)])
    colbp = jnp.concatenate([colb, jnp.full((pad,), TRASH, jnp.int32)])

    ekey = jnp.where(nonself, row * N + col, -1)
    skey = jnp.sort(ekey)
    rkey = col * N + row
    cnt = (jnp.searchsorted(skey, rkey, side='right')
           - jnp.searchsorted(skey, rkey, side='left')).astype(jnp.float32)
    cnt = jnp.where(nonself, cnt, 0.0)
    val128 = jnp.broadcast_to(jnp.pad(cnt, (0, pad))[:, None], (E_PAD, D))

    row_v = rowp.reshape(NW, EPT)
    col_v = colp.reshape(NW, EPT)
    colb_v = colbp.reshape(NW, EPT)

    x_pad = jnp.pad(x, ((0, NP - N), (0, 0)))
    zeros_nd = jnp.zeros((NP, D), jnp.float32)
    ones_nd = jnp.ones((NP, D), jnp.float32)
    eidx_v = jnp.arange(E_PAD, dtype=jnp.int32).reshape(NW, EPT)

    d2parts = _sc_edge_pass(val128, eidx_v, row_v, zeros_nd)
    degparts = _sc_edge_pass(ones_nd, row_v, col_v, zeros_nd)

    p1 = params['u1']
    hd, dis16, d2_16 = _k0(x_pad, d2parts, degparts, p1['down0_W'])

    ps = [params['u1'], params['u2'], params['u3'], params['u4']]
    for li, p in enumerate(ps):
        b0 = p['down0_b'].reshape(1, D)
        pw = p['pool_w'].reshape(1, D)
        b1 = p['down1_b'].reshape(1, D)
        b2 = p['up_b'].reshape(1, D)

        sparts = _sc_edge_pass(hd, row_v, col_v, zeros_nd)
        x1, sc16, key16 = _k1(sparts, hd, dis16, b0, pw)
        m128 = _k2(key16)
        t1mparts = _sc_edge_pass(m128, row_v, colb_v, zeros_nd)
        t1m = _k3b(t1mparts)
        t2mparts = _sc_edge_pass(t1m, row_v, colb_v, zeros_nd)
        hn, dist16 = _k3(x1, sc16, m128, t1m, t2mparts, d2_16, p['down1_W'])
        t1parts = _sc_edge_pass(hn, row_v, colb_v, zeros_nd)
        t1 = _k3b(t1parts)
        t2parts = _sc_edge_pass(t1, row_v, colb_v, zeros_nd)
        hd2 = _k4(t2parts, t1, hn, dist16, m128, d2_16, x1, dis16,
                  p['up_W'], b1)
        s2parts = _sc_edge_pass(hd2, row_v, col_v, zeros_nd)
        if li < 3:
            hd = _k5(s2parts, hd2, dis16, b2, ps[li + 1]['down0_W'])
        else:
            out = _k5f(s2parts, hd2, dis16, b2)

    return out[:N]


# speedup vs baseline: 2.1347x; 1.1521x over previous
import functools

import jax
import jax.numpy as jnp
from jax import lax
from jax.experimental import pallas as pl
from jax.experimental.pallas import tpu as pltpu
from jax.experimental.pallas import tpu_sc as plsc

N = 10000
NE = 160000
D = 128
NP = 10240
TRASH = N
K_POOL = (N + 1) // 2
NC, NS = 2, 16
NW = NC * NS
CHUNK = 128
E_PAD = 163840
EPT = E_PAD // NW
EPT1 = E_PAD // NS
BLK = 1024
GRID = NP // BLK

_mesh_cache = {}


def _vmesh():
    if "m" not in _mesh_cache:
        _mesh_cache["m"] = plsc.VectorSubcoreMesh(
            core_axis_name="c", subcore_axis_name="s",
            num_cores=NC, num_subcores=NS)
    return _mesh_cache["m"]


def _f32(*shape):
    return jax.ShapeDtypeStruct(shape, jnp.float32)


def _i32(*shape):
    return jax.ShapeDtypeStruct(shape, jnp.int32)


NBUF = 2


def _sc_edge_pass_body(table, rows_h, cols_h, zeros_h, out, acc, *scr):
    ridx = scr[0:NBUF]
    cidx = scr[NBUF:2 * NBUF]
    gbuf = scr[2 * NBUF:3 * NBUF]
    sem = scr[3 * NBUF:4 * NBUF]
    c = lax.axis_index("c")
    s = lax.axis_index("s")
    wid = s * NC + c
    rpt = NP // NS
    base = s * rpt
    pltpu.sync_copy(zeros_h.at[pl.ds(base, rpt)], acc.at[pl.ds(base, rpt)])
    plsc.subcore_barrier()

    nchunks = EPT // CHUNK

    def fire(j, b):
        pltpu.sync_copy(rows_h.at[wid, pl.ds(j * CHUNK, CHUNK)], ridx[b])
        pltpu.sync_copy(cols_h.at[wid, pl.ds(j * CHUNK, CHUNK)], cidx[b])
        pltpu.async_copy(table.at[ridx[b]], gbuf[b], sem[b])

    for b in range(NBUF):
        fire(b, b)

    def group(g, carry):
        j0 = g * NBUF
        for b in range(NBUF):
            j = j0 + b
            pltpu.make_async_copy(table.at[ridx[b]], gbuf[b], sem[b]).wait()
            pltpu.sync_copy(gbuf[b], acc.at[cidx[b]], add=True)

            @pl.when(j + NBUF < nchunks)
            def _():
                fire(j + NBUF, b)
        return carry

    lax.fori_loop(0, nchunks // NBUF, group, 0)
    plsc.subcore_barrier()
    pltpu.sync_copy(acc.at[pl.ds(base, rpt)], out.at[c, pl.ds(base, rpt)])


def _sc_edge_pass(table, rows, cols, zeros_nd):
    if "edge" not in _mesh_cache:
        _mesh_cache["edge"] = pl.kernel(
            _sc_edge_pass_body,
            out_type=_f32(NC, NP, D),
            mesh=_vmesh(),
            scratch_types=(
                [pltpu.VMEM_SHARED((NP, D), jnp.float32)]
                + [pltpu.VMEM((CHUNK,), jnp.int32) for _ in range(2 * NBUF)]
                + [pltpu.VMEM((CHUNK, D), jnp.float32) for _ in range(NBUF)]
                + [pltpu.SemaphoreType.DMA for _ in range(NBUF)]
            ),
        )
    return _mesh_cache["edge"](table, rows, cols, zeros_nd)


_ROWSPEC = pl.BlockSpec((BLK, D), lambda i: (i, 0))
_S16SPEC = pl.BlockSpec((BLK, 16), lambda i: (i, 0))
_P16SPEC = pl.BlockSpec((NC, BLK, 16), lambda i: (0, i, 0))
_PARTSPEC = pl.BlockSpec((NC, BLK, D), lambda i: (0, i, 0))
_WSPEC = pl.BlockSpec((D, D), lambda i: (0, 0))
_BSPEC = pl.BlockSpec((1, D), lambda i: (0, 0))


def _k0_body(x_ref, ap_ref, bp_ref, w0_ref, hd_ref, dis_ref, d2_ref):
    d2 = ap_ref[0, :, 0:1] + ap_ref[1, :, 0:1]
    deg = bp_ref[0, :, 0:1] + bp_ref[1, :, 0:1] + 1.0
    dis = lax.rsqrt(deg)
    colmask = lax.broadcasted_iota(jnp.int32, (BLK, 16), 1) == 0
    dis_ref[...] = jnp.where(colmask, dis, 0.0)
    d2_ref[...] = jnp.where(colmask, d2, 0.0)
    hd_ref[...] = dis * jnp.dot(x_ref[...], w0_ref[...],
                                preferred_element_type=jnp.float32)


_k0 = pl.pallas_call(
    _k0_body,
    grid=(GRID,),
    in_specs=[_ROWSPEC, _PARTSPEC, _PARTSPEC, _WSPEC],
    out_specs=[_ROWSPEC, _S16SPEC, _S16SPEC],
    out_shape=[_f32(NP, D), _f32(NP, 16), _f32(NP, 16)],
)


def _k1_body(sp_ref, hd_ref, dis_ref, b0_ref, pw_ref, x1_ref, sc_ref, key_ref):
    i = pl.program_id(0)
    dis = dis_ref[:, 0:1]
    agg = sp_ref[0, :, :] + sp_ref[1, :, :] + hd_ref[...]
    x1 = jnp.maximum(dis * agg + b0_ref[...], 0.0)
    x1_ref[...] = x1
    pw = pw_ref[...]
    nrm = jnp.sqrt(jnp.sum(pw * pw))
    s = jnp.tanh(jnp.sum(x1 * pw, axis=1, keepdims=True) / nrm)
    colmask = lax.broadcasted_iota(jnp.int32, (BLK, 16), 1) == 0
    sc_ref[...] = jnp.where(colmask, s, 0.0)
    b = lax.bitcast_convert_type(s, jnp.int32)
    key = jnp.where(b >= 0, b + jnp.int32(0x40000000),
                    jnp.int32(0x40000000) - (b & jnp.int32(0x7FFFFFFF)))
    node = i * BLK + lax.broadcasted_iota(jnp.int32, (BLK, 16), 0)
    valid = (node < N) & colmask
    key_ref[...] = jnp.where(valid, key, jnp.int32(0))


_k1 = pl.pallas_call(
    _k1_body,
    grid=(GRID,),
    in_specs=[_PARTSPEC, _ROWSPEC, _S16SPEC, _BSPEC, _BSPEC],
    out_specs=[_ROWSPEC, _S16SPEC, _S16SPEC],
    out_shape=[_f32(NP, D), _f32(NP, 16), _i32(NP, 16)],
)


def _k2_body(key_ref, m_ref):
    keys = key_ref[...]
    idx = lax.broadcasted_iota(jnp.int32, (NP, 16), 0)
    col0 = lax.broadcasted_iota(jnp.int32, (NP, 16), 1) == 0
    kk = jnp.int32(K_POOL)

    def bit_step(i, t):
        cand = t | (jnp.int32(1) << (jnp.int32(30) - i))
        cnt = jnp.sum((keys >= cand).astype(jnp.int32))
        return jnp.where(cnt >= kk, cand, t)

    thr = lax.fori_loop(0, 31, bit_step, jnp.int32(0))
    cgt = jnp.sum((keys > thr).astype(jnp.int32))
    r = kk - cgt
    eq = (keys == thr) & col0

    def idx_step(i, a):
        cand = a + (jnp.int32(1) << (jnp.int32(13) - i))
        cnt = jnp.sum((eq & (idx <= cand - 1)).astype(jnp.int32))
        return jnp.where(cnt < r, cand, a)

    a = lax.fori_loop(0, 14, idx_step, jnp.int32(0))
    sel = (keys > thr) | (eq & (idx <= a))
    msel = jnp.max(sel.astype(jnp.float32), axis=1, keepdims=True)
    m_ref[...] = jnp.broadcast_to(msel, (NP, D))


_k2 = pl.pallas_call(
    _k2_body,
    grid=(1,),
    in_specs=[pl.BlockSpec((NP, 16), lambda i: (0, 0))],
    out_specs=pl.BlockSpec((NP, D), lambda i: (0, 0)),
    out_shape=_f32(NP, D),
)


def _k3_body(x1_ref, sc_ref, m_ref, t1_ref, t2p_ref, d2_ref, w1_ref,
             hn_ref, dist_ref):
    m = m_ref[:, 0:1]
    t2m = t2p_ref[0, :, 0:1] + t2p_ref[1, :, 0:1]
    dega = t2m + 2.0 * t1_ref[:, 0:1] - d2_ref[:, 0:1] * m
    dist = m * lax.rsqrt(1.0 + dega)
    colmask = lax.broadcasted_iota(jnp.int32, (BLK, 16), 1) == 0
    dist_ref[...] = jnp.where(colmask, dist, 0.0)
    xt = (m * sc_ref[:, 0:1]) * x1_ref[...]
    hn_ref[...] = dist * jnp.dot(xt, w1_ref[...],
                                 preferred_element_type=jnp.float32)


_k3 = pl.pallas_call(
    _k3_body,
    grid=(GRID,),
    in_specs=[_ROWSPEC, _S16SPEC, _ROWSPEC, _ROWSPEC, _PARTSPEC, _S16SPEC,
              _WSPEC],
    out_specs=[_ROWSPEC, _S16SPEC],
    out_shape=[_f32(NP, D), _f32(NP, 16)],
)


def _k3b_body(tp_ref, t1_ref):
    t1_ref[...] = tp_ref[0, :, :] + tp_ref[1, :, :]


_k3b = pl.pallas_call(
    _k3b_body,
    grid=(GRID,),
    in_specs=[_PARTSPEC],
    out_specs=_ROWSPEC,
    out_shape=_f32(NP, D),
)


def _k4_body(t2p_ref, t1_ref, hn_ref, dist_ref, m_ref, d2_ref, x1_ref,
             dis_ref, w2_ref, b1_ref, hd2_ref):
    hn = hn_ref[...]
    y = t2p_ref[0, :, :] + t2p_ref[1, :, :] + 2.0 * t1_ref[...] \
        - d2_ref[:, 0:1] * hn
    x2 = m_ref[:, 0:1] * jnp.maximum(dist_ref[:, 0:1] * (y + hn) + b1_ref[...],
                                     0.0)
    z = x1_ref[...] + x2
    hd2_ref[...] = dis_ref[:, 0:1] * jnp.dot(z, w2_ref[...],
                                             preferred_element_type=jnp.float32)


_k4 = pl.pallas_call(
    _k4_body,
    grid=(GRID,),
    in_specs=[_PARTSPEC, _ROWSPEC, _ROWSPEC, _S16SPEC, _ROWSPEC, _S16SPEC,
              _ROWSPEC, _S16SPEC, _WSPEC, _BSPEC],
    out_specs=_ROWSPEC,
    out_shape=_f32(NP, D),
)


def _k5_body(sp_ref, hd2_ref, dis_ref, b2_ref, w0n_ref, hdn_ref):
    dis = dis_ref[:, 0:1]
    o = dis * (sp_ref[0, :, :] + sp_ref[1, :, :] + hd2_ref[...]) + b2_ref[...]
    xn = jnp.where(o > 0, o, jnp.exp(o) - 1.0)
    hdn_ref[...] = dis * jnp.dot(xn, w0n_ref[...],
                                 preferred_element_type=jnp.float32)


_k5 = pl.pallas_call(
    _k5_body,
    grid=(GRID,),
    in_specs=[_PARTSPEC, _ROWSPEC, _S16SPEC, _BSPEC, _WSPEC],
    out_specs=_ROWSPEC,
    out_shape=_f32(NP, D),
)


def _k5f_body(sp_ref, hd2_ref, dis_ref, b2_ref, out_ref):
    dis = dis_ref[:, 0:1]
    o = dis * (sp_ref[0, :, :] + sp_ref[1, :, :] + hd2_ref[...]) + b2_ref[...]
    out_ref[...] = jnp.where(o > 0, o, float(D) * (jnp.exp(o) - 1.0))


_k5f = pl.pallas_call(
    _k5f_body,
    grid=(GRID,),
    in_specs=[_PARTSPEC, _ROWSPEC, _S16SPEC, _BSPEC],
    out_specs=_ROWSPEC,
    out_shape=_f32(NP, D),
)


def kernel(x, edge_index, edge_attr, params):
    row = edge_index[0].astype(jnp.int32)
    col = edge_index[1].astype(jnp.int32)
    nonself = row != col
    colb = jnp.where(nonself, col, TRASH)

    pad = E_PAD - NE
    rowp = jnp.concatenate([row, jnp.zeros((pad,), jnp.int32)])
    colp = jnp.concatenate([col, jnp.full((pad,), TRASH, jnp.int32)])
    colbp = jnp.concatenate([colb, jnp.full((pad,), TRASH, jnp.int32)])

    ekey = jnp.where(nonself, row * N + col, -1)
    skey = jnp.sort(ekey)
    rkey = col * N + row
    cnt = (jnp.searchsorted(skey, rkey, side='right')
           - jnp.searchsorted(skey, rkey, side='left')).astype(jnp.float32)
    cnt = jnp.where(nonself, cnt, 0.0)
    val128 = jnp.broadcast_to(jnp.pad(cnt, (0, pad))[:, None], (E_PAD, D))

    row_v = rowp.reshape(NW, EPT)
    col_v = colp.reshape(NW, EPT)
    colb_v = colbp.reshape(NW, EPT)

    x_pad = jnp.pad(x, ((0, NP - N), (0, 0)))
    zeros_nd = jnp.zeros((NP, D), jnp.float32)
    ones_nd = jnp.ones((NP, D), jnp.float32)
    eidx_v = jnp.arange(E_PAD, dtype=jnp.int32).reshape(NW, EPT)

    d2parts = _sc_edge_pass(val128, eidx_v, row_v, zeros_nd)
    degparts = _sc_edge_pass(ones_nd, row_v, col_v, zeros_nd)

    p1 = params['u1']
    hd, dis16, d2_16 = _k0(x_pad, d2parts, degparts, p1['down0_W'])

    ps = [params['u1'], params['u2'], params['u3'], params['u4']]
    for li, p in enumerate(ps):
        b0 = p['down0_b'].reshape(1, D)
        pw = p['pool_w'].reshape(1, D)
        b1 = p['down1_b'].reshape(1, D)
        b2 = p['up_b'].reshape(1, D)

        sparts = _sc_edge_pass(hd, row_v, col_v, zeros_nd)
        x1, sc16, key16 = _k1(sparts, hd, dis16, b0, pw)
        m128 = _k2(key16)
        t1mparts = _sc_edge_pass(m128, row_v, colb_v, zeros_nd)
        t1m = _k3b(t1mparts)
        t2mparts = _sc_edge_pass(t1m, row_v, colb_v, zeros_nd)
        hn, dist16 = _k3(x1, sc16, m128, t1m, t2mparts, d2_16, p['down1_W'])
        t1parts = _sc_edge_pass(hn, row_v, colb_v, zeros_nd)
        t1 = _k3b(t1parts)
        t2parts = _sc_edge_pass(t1, row_v, colb_v, zeros_nd)
        hd2 = _k4(t2parts, t1, hn, dist16, m128, d2_16, x1, dis16,
                  p['up_W'], b1)
        s2parts = _sc_edge_pass(hd2, row_v, col_v, zeros_nd)
        if li < 3:
            hd = _k5(s2parts, hd2, dis16, b2, ps[li + 1]['down0_W'])
        else:
            out = _k5f(s2parts, hd2, dis16, b2)

    return out[:N]


# speedup vs baseline: 2.3730x; 1.1116x over previous
import functools

import jax
import jax.numpy as jnp
from jax import lax
from jax.experimental import pallas as pl
from jax.experimental.pallas import tpu as pltpu
from jax.experimental.pallas import tpu_sc as plsc

N = 10000
NE = 160000
D = 128
NP = 10240
TRASH = N
K_POOL = (N + 1) // 2
NC, NS = 2, 16
NW = NC * NS
CHUNK = 64
E_PAD = 163840
EPT = E_PAD // NW
EPT1 = E_PAD // NS
BLK = 1024
GRID = NP // BLK

_mesh_cache = {}


def _vmesh():
    if "m" not in _mesh_cache:
        _mesh_cache["m"] = plsc.VectorSubcoreMesh(
            core_axis_name="c", subcore_axis_name="s",
            num_cores=NC, num_subcores=NS)
    return _mesh_cache["m"]


def _f32(*shape):
    return jax.ShapeDtypeStruct(shape, jnp.float32)


def _i32(*shape):
    return jax.ShapeDtypeStruct(shape, jnp.int32)


NBUF = 4
NCH = EPT // CHUNK


def _sc_edge_pass_body(table, rows_h, cols3_h, zeros_h, out, acc, rstage,
                       cstage, *scr):
    gbuf = scr[0:NBUF]
    gsem = scr[NBUF:2 * NBUF]
    ssem = scr[2 * NBUF:3 * NBUF]
    c = lax.axis_index("c")
    s = lax.axis_index("s")
    wid = s * NC + c
    rpt = NP // NS
    base = s * rpt
    pltpu.sync_copy(rows_h.at[wid], rstage)
    pltpu.sync_copy(cols3_h.at[wid], cstage)
    pltpu.sync_copy(zeros_h.at[pl.ds(base, rpt)], acc.at[pl.ds(base, rpt)])
    plsc.subcore_barrier()

    def fire_gather(j, b):
        pltpu.async_copy(table.at[rstage.at[pl.ds(j * CHUNK, CHUNK)]],
                         gbuf[b], gsem[b])

    for b in range(NBUF - 1):
        fire_gather(b, b)

    def group(g, carry):
        j0 = g * NBUF
        for b in range(NBUF):
            j = j0 + b
            pltpu.make_async_copy(
                table.at[rstage.at[pl.ds(j * CHUNK, CHUNK)]],
                gbuf[b], gsem[b]).wait()
            pltpu.async_copy(gbuf[b], acc.at[cstage.at[j]], ssem[b], add=True)
            bp = (b + NBUF - 1) % NBUF
            jp = j + NBUF - 1

            @pl.when(jp < NCH)
            def _():
                @pl.when(jp >= NBUF)
                def _():
                    pltpu.make_async_copy(gbuf[bp], acc.at[cstage.at[0]],
                                          ssem[bp]).wait()
                fire_gather(jp, bp)
        return carry

    lax.fori_loop(0, NCH // NBUF, group, 0)
    for b in range(NBUF):
        pltpu.make_async_copy(gbuf[b], acc.at[cstage.at[0]], ssem[b]).wait()
    plsc.subcore_barrier()
    pltpu.sync_copy(acc.at[pl.ds(base, rpt)], out.at[c, pl.ds(base, rpt)])


def _sc_edge_pass(table, rows, cols, zeros_nd):
    if "edge" not in _mesh_cache:
        _mesh_cache["edge"] = pl.kernel(
            _sc_edge_pass_body,
            out_type=_f32(NC, NP, D),
            mesh=_vmesh(),
            scratch_types=(
                [pltpu.VMEM_SHARED((NP, D), jnp.float32),
                 pltpu.VMEM((EPT,), jnp.int32),
                 pltpu.VMEM((NCH, CHUNK), jnp.int32)]
                + [pltpu.VMEM((CHUNK, D), jnp.float32) for _ in range(NBUF)]
                + [pltpu.SemaphoreType.DMA for _ in range(2 * NBUF)]
            ),
        )
    return _mesh_cache["edge"](table, rows, cols, zeros_nd)


_ROWSPEC = pl.BlockSpec((BLK, D), lambda i: (i, 0))
_S16SPEC = pl.BlockSpec((BLK, 16), lambda i: (i, 0))
_P16SPEC = pl.BlockSpec((NC, BLK, 16), lambda i: (0, i, 0))
_PARTSPEC = pl.BlockSpec((NC, BLK, D), lambda i: (0, i, 0))
_WSPEC = pl.BlockSpec((D, D), lambda i: (0, 0))
_BSPEC = pl.BlockSpec((1, D), lambda i: (0, 0))


def _k0_body(x_ref, ap_ref, bp_ref, w0_ref, hd_ref, dis_ref, d2_ref):
    d2 = ap_ref[0, :, 0:1] + ap_ref[1, :, 0:1]
    deg = bp_ref[0, :, 0:1] + bp_ref[1, :, 0:1] + 1.0
    dis = lax.rsqrt(deg)
    colmask = lax.broadcasted_iota(jnp.int32, (BLK, 16), 1) == 0
    dis_ref[...] = jnp.where(colmask, dis, 0.0)
    d2_ref[...] = jnp.where(colmask, d2, 0.0)
    hd_ref[...] = dis * jnp.dot(x_ref[...], w0_ref[...],
                                preferred_element_type=jnp.float32)


_k0 = pl.pallas_call(
    _k0_body,
    grid=(GRID,),
    in_specs=[_ROWSPEC, _PARTSPEC, _PARTSPEC, _WSPEC],
    out_specs=[_ROWSPEC, _S16SPEC, _S16SPEC],
    out_shape=[_f32(NP, D), _f32(NP, 16), _f32(NP, 16)],
)


def _k1_body(sp_ref, hd_ref, dis_ref, b0_ref, pw_ref, x1_ref, sc_ref, key_ref):
    i = pl.program_id(0)
    dis = dis_ref[:, 0:1]
    agg = sp_ref[0, :, :] + sp_ref[1, :, :] + hd_ref[...]
    x1 = jnp.maximum(dis * agg + b0_ref[...], 0.0)
    x1_ref[...] = x1
    pw = pw_ref[...]
    nrm = jnp.sqrt(jnp.sum(pw * pw))
    s = jnp.tanh(jnp.sum(x1 * pw, axis=1, keepdims=True) / nrm)
    colmask = lax.broadcasted_iota(jnp.int32, (BLK, 16), 1) == 0
    sc_ref[...] = jnp.where(colmask, s, 0.0)
    b = lax.bitcast_convert_type(s, jnp.int32)
    key = jnp.where(b >= 0, b + jnp.int32(0x40000000),
                    jnp.int32(0x40000000) - (b & jnp.int32(0x7FFFFFFF)))
    node = i * BLK + lax.broadcasted_iota(jnp.int32, (BLK, 16), 0)
    valid = (node < N) & colmask
    key_ref[...] = jnp.where(valid, key, jnp.int32(0))


_k1 = pl.pallas_call(
    _k1_body,
    grid=(GRID,),
    in_specs=[_PARTSPEC, _ROWSPEC, _S16SPEC, _BSPEC, _BSPEC],
    out_specs=[_ROWSPEC, _S16SPEC, _S16SPEC],
    out_shape=[_f32(NP, D), _f32(NP, 16), _i32(NP, 16)],
)


def _k2_body(key_ref, m_ref):
    keys = key_ref[...]
    idx = lax.broadcasted_iota(jnp.int32, (NP, 16), 0)
    col0 = lax.broadcasted_iota(jnp.int32, (NP, 16), 1) == 0
    kk = jnp.int32(K_POOL)

    def bit_step(i, t):
        cand = t | (jnp.int32(1) << (jnp.int32(30) - i))
        cnt = jnp.sum((keys >= cand).astype(jnp.int32))
        return jnp.where(cnt >= kk, cand, t)

    thr = lax.fori_loop(0, 31, bit_step, jnp.int32(0))
    cgt = jnp.sum((keys > thr).astype(jnp.int32))
    r = kk - cgt
    eq = (keys == thr) & col0

    def idx_step(i, a):
        cand = a + (jnp.int32(1) << (jnp.int32(13) - i))
        cnt = jnp.sum((eq & (idx <= cand - 1)).astype(jnp.int32))
        return jnp.where(cnt < r, cand, a)

    a = lax.fori_loop(0, 14, idx_step, jnp.int32(0))
    sel = (keys > thr) | (eq & (idx <= a))
    msel = jnp.max(sel.astype(jnp.float32), axis=1, keepdims=True)
    m_ref[...] = jnp.broadcast_to(msel, (NP, D))


_k2 = pl.pallas_call(
    _k2_body,
    grid=(1,),
    in_specs=[pl.BlockSpec((NP, 16), lambda i: (0, 0))],
    out_specs=pl.BlockSpec((NP, D), lambda i: (0, 0)),
    out_shape=_f32(NP, D),
)


def _k3_body(x1_ref, sc_ref, m_ref, t1_ref, t2p_ref, d2_ref, w1_ref,
             hn_ref, dist_ref):
    m = m_ref[:, 0:1]
    t2m = t2p_ref[0, :, 0:1] + t2p_ref[1, :, 0:1]
    dega = t2m + 2.0 * t1_ref[:, 0:1] - d2_ref[:, 0:1] * m
    dist = m * lax.rsqrt(1.0 + dega)
    colmask = lax.broadcasted_iota(jnp.int32, (BLK, 16), 1) == 0
    dist_ref[...] = jnp.where(colmask, dist, 0.0)
    xt = (m * sc_ref[:, 0:1]) * x1_ref[...]
    hn_ref[...] = dist * jnp.dot(xt, w1_ref[...],
                                 preferred_element_type=jnp.float32)


_k3 = pl.pallas_call(
    _k3_body,
    grid=(GRID,),
    in_specs=[_ROWSPEC, _S16SPEC, _ROWSPEC, _ROWSPEC, _PARTSPEC, _S16SPEC,
              _WSPEC],
    out_specs=[_ROWSPEC, _S16SPEC],
    out_shape=[_f32(NP, D), _f32(NP, 16)],
)


def _k3b_body(tp_ref, t1_ref):
    t1_ref[...] = tp_ref[0, :, :] + tp_ref[1, :, :]


_k3b = pl.pallas_call(
    _k3b_body,
    grid=(GRID,),
    in_specs=[_PARTSPEC],
    out_specs=_ROWSPEC,
    out_shape=_f32(NP, D),
)


def _k4_body(t2p_ref, t1_ref, hn_ref, dist_ref, m_ref, d2_ref, x1_ref,
             dis_ref, w2_ref, b1_ref, hd2_ref):
    hn = hn_ref[...]
    y = t2p_ref[0, :, :] + t2p_ref[1, :, :] + 2.0 * t1_ref[...] \
        - d2_ref[:, 0:1] * hn
    x2 = m_ref[:, 0:1] * jnp.maximum(dist_ref[:, 0:1] * (y + hn) + b1_ref[...],
                                     0.0)
    z = x1_ref[...] + x2
    hd2_ref[...] = dis_ref[:, 0:1] * jnp.dot(z, w2_ref[...],
                                             preferred_element_type=jnp.float32)


_k4 = pl.pallas_call(
    _k4_body,
    grid=(GRID,),
    in_specs=[_PARTSPEC, _ROWSPEC, _ROWSPEC, _S16SPEC, _ROWSPEC, _S16SPEC,
              _ROWSPEC, _S16SPEC, _WSPEC, _BSPEC],
    out_specs=_ROWSPEC,
    out_shape=_f32(NP, D),
)


def _k5_body(sp_ref, hd2_ref, dis_ref, b2_ref, w0n_ref, hdn_ref):
    dis = dis_ref[:, 0:1]
    o = dis * (sp_ref[0, :, :] + sp_ref[1, :, :] + hd2_ref[...]) + b2_ref[...]
    xn = jnp.where(o > 0, o, jnp.exp(o) - 1.0)
    hdn_ref[...] = dis * jnp.dot(xn, w0n_ref[...],
                                 preferred_element_type=jnp.float32)


_k5 = pl.pallas_call(
    _k5_body,
    grid=(GRID,),
    in_specs=[_PARTSPEC, _ROWSPEC, _S16SPEC, _BSPEC, _WSPEC],
    out_specs=_ROWSPEC,
    out_shape=_f32(NP, D),
)


def _k5f_body(sp_ref, hd2_ref, dis_ref, b2_ref, out_ref):
    dis = dis_ref[:, 0:1]
    o = dis * (sp_ref[0, :, :] + sp_ref[1, :, :] + hd2_ref[...]) + b2_ref[...]
    out_ref[...] = jnp.where(o > 0, o, float(D) * (jnp.exp(o) - 1.0))


_k5f = pl.pallas_call(
    _k5f_body,
    grid=(GRID,),
    in_specs=[_PARTSPEC, _ROWSPEC, _S16SPEC, _BSPEC],
    out_specs=_ROWSPEC,
    out_shape=_f32(NP, D),
)


def kernel(x, edge_index, edge_attr, params):
    row = edge_index[0].astype(jnp.int32)
    col = edge_index[1].astype(jnp.int32)
    nonself = row != col
    colb = jnp.where(nonself, col, TRASH)

    pad = E_PAD - NE
    rowp = jnp.concatenate([row, jnp.zeros((pad,), jnp.int32)])
    colp = jnp.concatenate([col, jnp.full((pad,), TRASH, jnp.int32)])
    colbp = jnp.concatenate([colb, jnp.full((pad,), TRASH, jnp.int32)])

    ekey = jnp.where(nonself, row * N + col, -1)
    rkey = col * N + row
    tagged = jnp.concatenate([ekey * 2, rkey * 2 + 1])
    payload = jnp.arange(2 * NE, dtype=jnp.int32)
    skey, sidx = lax.sort_key_val(tagged, payload)
    key = skey >> 1
    is0 = (skey & 1) == 0
    newrun = jnp.concatenate(
        [jnp.ones((1,), jnp.bool_), key[1:] != key[:-1]])
    rid = jnp.cumsum(newrun.astype(jnp.int32)) - 1
    cnt_run = jnp.zeros((2 * NE,), jnp.float32).at[rid].add(
        is0.astype(jnp.float32))
    cntpos = cnt_run[rid]
    back = jnp.where(is0, 0, sidx - NE)
    cnt = jnp.zeros((NE,), jnp.float32).at[back].add(
        jnp.where(is0, 0.0, cntpos))
    cnt = jnp.where(nonself, cnt, 0.0)
    val128 = jnp.broadcast_to(jnp.pad(cnt, (0, pad))[:, None], (E_PAD, D))

    row_v = rowp.reshape(NW, EPT)
    row_v3 = rowp.reshape(NW, NCH, CHUNK)
    col_v3 = colp.reshape(NW, NCH, CHUNK)
    colb_v3 = colbp.reshape(NW, NCH, CHUNK)

    x_pad = jnp.pad(x, ((0, NP - N), (0, 0)))
    zeros_nd = jnp.zeros((NP, D), jnp.float32)
    ones_nd = jnp.ones((NP, D), jnp.float32)
    eidx_v = jnp.arange(E_PAD, dtype=jnp.int32).reshape(NW, EPT)

    d2parts = _sc_edge_pass(val128, eidx_v, row_v3, zeros_nd)
    degparts = _sc_edge_pass(ones_nd, row_v, col_v3, zeros_nd)

    p1 = params['u1']
    hd, dis16, d2_16 = _k0(x_pad, d2parts, degparts, p1['down0_W'])

    ps = [params['u1'], params['u2'], params['u3'], params['u4']]
    for li, p in enumerate(ps):
        b0 = p['down0_b'].reshape(1, D)
        pw = p['pool_w'].reshape(1, D)
        b1 = p['down1_b'].reshape(1, D)
        b2 = p['up_b'].reshape(1, D)

        sparts = _sc_edge_pass(hd, row_v, col_v3, zeros_nd)
        x1, sc16, key16 = _k1(sparts, hd, dis16, b0, pw)
        m128 = _k2(key16)
        t1mparts = _sc_edge_pass(m128, row_v, colb_v3, zeros_nd)
        t1m = _k3b(t1mparts)
        t2mparts = _sc_edge_pass(t1m, row_v, colb_v3, zeros_nd)
        hn, dist16 = _k3(x1, sc16, m128, t1m, t2mparts, d2_16, p['down1_W'])
        t1parts = _sc_edge_pass(hn, row_v, colb_v3, zeros_nd)
        t1 = _k3b(t1parts)
        t2parts = _sc_edge_pass(t1, row_v, colb_v3, zeros_nd)
        hd2 = _k4(t2parts, t1, hn, dist16, m128, d2_16, x1, dis16,
                  p['up_W'], b1)
        s2parts = _sc_edge_pass(hd2, row_v, col_v3, zeros_nd)
        if li < 3:
            hd = _k5(s2parts, hd2, dis16, b2, ps[li + 1]['down0_W'])
        else:
            out = _k5f(s2parts, hd2, dis16, b2)

    return out[:N]


# speedup vs baseline: 2.9765x; 1.2543x over previous
import functools

import jax
import jax.numpy as jnp
from jax import lax
from jax.experimental import pallas as pl
from jax.experimental.pallas import tpu as pltpu
from jax.experimental.pallas import tpu_sc as plsc

N = 10000
NE = 160000
D = 128
NP = 10240
TRASH = N
K_POOL = (N + 1) // 2
NC, NS = 2, 16
NW = NC * NS
CHUNK = 64
E_PAD = 163840
EPT = E_PAD // NW
EPT1 = E_PAD // NS
BLK = 1024
GRID = NP // BLK

_mesh_cache = {}


def _vmesh():
    if "m" not in _mesh_cache:
        _mesh_cache["m"] = plsc.VectorSubcoreMesh(
            core_axis_name="c", subcore_axis_name="s",
            num_cores=NC, num_subcores=NS)
    return _mesh_cache["m"]


def _f32(*shape):
    return jax.ShapeDtypeStruct(shape, jnp.float32)


def _i32(*shape):
    return jax.ShapeDtypeStruct(shape, jnp.int32)


NBUF = 4
NCH = EPT // CHUNK


def _make_edge_pass(ept, chunk, nbuf):
    nch = ept // chunk

    def body(table, rows_h, cols3_h, zeros_h, out, acc, rstage, cstage, *scr):
        gbuf = scr[0:nbuf]
        gsem = scr[nbuf:2 * nbuf]
        ssem = scr[2 * nbuf:3 * nbuf]
        c = lax.axis_index("c")
        s = lax.axis_index("s")
        wid = s * NC + c
        rpt = NP // NS
        base = s * rpt
        pltpu.sync_copy(rows_h.at[wid], rstage)
        pltpu.sync_copy(cols3_h.at[wid], cstage)
        pltpu.sync_copy(zeros_h.at[pl.ds(base, rpt)], acc.at[pl.ds(base, rpt)])
        plsc.subcore_barrier()

        def fire_gather(j, b):
            pltpu.async_copy(table.at[rstage.at[pl.ds(j * chunk, chunk)]],
                             gbuf[b], gsem[b])

        for b in range(nbuf - 1):
            fire_gather(b, b)

        def group(g, carry):
            j0 = g * nbuf
            for b in range(nbuf):
                j = j0 + b
                pltpu.make_async_copy(
                    table.at[rstage.at[pl.ds(j * chunk, chunk)]],
                    gbuf[b], gsem[b]).wait()
                pltpu.async_copy(gbuf[b], acc.at[cstage.at[j]], ssem[b],
                                 add=True)
                bp = (b + nbuf - 1) % nbuf
                jp = j + nbuf - 1

                @pl.when(jp < nch)
                def _():
                    @pl.when(jp >= nbuf)
                    def _():
                        pltpu.make_async_copy(gbuf[bp], acc.at[cstage.at[0]],
                                              ssem[bp]).wait()
                    fire_gather(jp, bp)
            return carry

        lax.fori_loop(0, nch // nbuf, group, 0)
        for b in range(nbuf):
            pltpu.make_async_copy(gbuf[b], acc.at[cstage.at[0]],
                                  ssem[b]).wait()
        plsc.subcore_barrier()
        pltpu.sync_copy(acc.at[pl.ds(base, rpt)], out.at[c, pl.ds(base, rpt)])

    return pl.kernel(
        body,
        out_type=_f32(NC, NP, D),
        mesh=_vmesh(),
        scratch_types=(
            [pltpu.VMEM_SHARED((NP, D), jnp.float32),
             pltpu.VMEM((ept,), jnp.int32),
             pltpu.VMEM((nch, chunk), jnp.int32)]
            + [pltpu.VMEM((chunk, D), jnp.float32) for _ in range(nbuf)]
            + [pltpu.SemaphoreType.DMA for _ in range(2 * nbuf)]
        ),
    )


def _sc_edge_pass(table, rows, cols, zeros_nd, chunk=CHUNK, nbuf=NBUF):
    key = ("edge", rows.shape[1], chunk, nbuf)
    if key not in _mesh_cache:
        _mesh_cache[key] = _make_edge_pass(rows.shape[1], chunk, nbuf)
    return _mesh_cache[key](table, rows, cols, zeros_nd)


_ROWSPEC = pl.BlockSpec((BLK, D), lambda i: (i, 0))
_S16SPEC = pl.BlockSpec((BLK, 16), lambda i: (i, 0))
_P16SPEC = pl.BlockSpec((NC, BLK, 16), lambda i: (0, i, 0))
_PARTSPEC = pl.BlockSpec((NC, BLK, D), lambda i: (0, i, 0))
_WSPEC = pl.BlockSpec((D, D), lambda i: (0, 0))
_BSPEC = pl.BlockSpec((1, D), lambda i: (0, 0))


def _k0_body(x_ref, apa_ref, apb_ref, bp_ref, w0_ref, hd_ref, dis_ref,
             d2_ref):
    d2 = (apa_ref[0, :, 0:1] + apa_ref[1, :, 0:1]
          + apb_ref[0, :, 0:1] + apb_ref[1, :, 0:1])
    deg = bp_ref[0, :, 0:1] + bp_ref[1, :, 0:1] + 1.0
    dis = lax.rsqrt(deg)
    colmask = lax.broadcasted_iota(jnp.int32, (BLK, 16), 1) == 0
    dis_ref[...] = jnp.where(colmask, dis, 0.0)
    d2_ref[...] = jnp.where(colmask, d2, 0.0)
    hd_ref[...] = dis * jnp.dot(x_ref[...], w0_ref[...],
                                preferred_element_type=jnp.float32)


_k0 = pl.pallas_call(
    _k0_body,
    grid=(GRID,),
    in_specs=[_ROWSPEC, _PARTSPEC, _PARTSPEC, _PARTSPEC, _WSPEC],
    out_specs=[_ROWSPEC, _S16SPEC, _S16SPEC],
    out_shape=[_f32(NP, D), _f32(NP, 16), _f32(NP, 16)],
)


def _k1_body(sp_ref, hd_ref, dis_ref, b0_ref, pw_ref, x1_ref, sc_ref, key_ref):
    i = pl.program_id(0)
    dis = dis_ref[:, 0:1]
    agg = sp_ref[0, :, :] + sp_ref[1, :, :] + hd_ref[...]
    x1 = jnp.maximum(dis * agg + b0_ref[...], 0.0)
    x1_ref[...] = x1
    pw = pw_ref[...]
    nrm = jnp.sqrt(jnp.sum(pw * pw))
    s = jnp.tanh(jnp.sum(x1 * pw, axis=1, keepdims=True) / nrm)
    colmask = lax.broadcasted_iota(jnp.int32, (BLK, 16), 1) == 0
    sc_ref[...] = jnp.where(colmask, s, 0.0)
    b = lax.bitcast_convert_type(s, jnp.int32)
    key = jnp.where(b >= 0, b + jnp.int32(0x40000000),
                    jnp.int32(0x40000000) - (b & jnp.int32(0x7FFFFFFF)))
    node = i * BLK + lax.broadcasted_iota(jnp.int32, (BLK, 16), 0)
    valid = (node < N) & colmask
    key_ref[...] = jnp.where(valid, key, jnp.int32(0))


_k1 = pl.pallas_call(
    _k1_body,
    grid=(GRID,),
    in_specs=[_PARTSPEC, _ROWSPEC, _S16SPEC, _BSPEC, _BSPEC],
    out_specs=[_ROWSPEC, _S16SPEC, _S16SPEC],
    out_shape=[_f32(NP, D), _f32(NP, 16), _i32(NP, 16)],
)


def _k2_body(key_ref, m_ref):
    keys = key_ref[...]
    idx = lax.broadcasted_iota(jnp.int32, (NP, 16), 0)
    col0 = lax.broadcasted_iota(jnp.int32, (NP, 16), 1) == 0
    kk = jnp.int32(K_POOL)

    def bit_step(i, t):
        cand = t | (jnp.int32(1) << (jnp.int32(30) - i))
        cnt = jnp.sum((keys >= cand).astype(jnp.int32))
        return jnp.where(cnt >= kk, cand, t)

    thr = lax.fori_loop(0, 31, bit_step, jnp.int32(0))
    cgt = jnp.sum((keys > thr).astype(jnp.int32))
    r = kk - cgt
    eq = (keys == thr) & col0

    def idx_step(i, a):
        cand = a + (jnp.int32(1) << (jnp.int32(13) - i))
        cnt = jnp.sum((eq & (idx <= cand - 1)).astype(jnp.int32))
        return jnp.where(cnt < r, cand, a)

    a = lax.fori_loop(0, 14, idx_step, jnp.int32(0))
    sel = (keys > thr) | (eq & (idx <= a))
    msel = jnp.max(sel.astype(jnp.float32), axis=1, keepdims=True)
    m_ref[...] = jnp.broadcast_to(msel, (NP, D))


_k2 = pl.pallas_call(
    _k2_body,
    grid=(1,),
    in_specs=[pl.BlockSpec((NP, 16), lambda i: (0, 0))],
    out_specs=pl.BlockSpec((NP, D), lambda i: (0, 0)),
    out_shape=_f32(NP, D),
)


def _k3_body(x1_ref, sc_ref, m_ref, t1_ref, t2p_ref, d2_ref, w1_ref,
             hn_ref, dist_ref):
    m = m_ref[:, 0:1]
    t2m = t2p_ref[0, :, 0:1] + t2p_ref[1, :, 0:1]
    dega = t2m + 2.0 * t1_ref[:, 0:1] - d2_ref[:, 0:1] * m
    dist = m * lax.rsqrt(1.0 + dega)
    colmask = lax.broadcasted_iota(jnp.int32, (BLK, 16), 1) == 0
    dist_ref[...] = jnp.where(colmask, dist, 0.0)
    xt = (m * sc_ref[:, 0:1]) * x1_ref[...]
    hn_ref[...] = dist * jnp.dot(xt, w1_ref[...],
                                 preferred_element_type=jnp.float32)


_k3 = pl.pallas_call(
    _k3_body,
    grid=(GRID,),
    in_specs=[_ROWSPEC, _S16SPEC, _ROWSPEC, _ROWSPEC, _PARTSPEC, _S16SPEC,
              _WSPEC],
    out_specs=[_ROWSPEC, _S16SPEC],
    out_shape=[_f32(NP, D), _f32(NP, 16)],
)


def _k3b_body(tp_ref, t1_ref):
    t1_ref[...] = tp_ref[0, :, :] + tp_ref[1, :, :]


_k3b = pl.pallas_call(
    _k3b_body,
    grid=(GRID,),
    in_specs=[_PARTSPEC],
    out_specs=_ROWSPEC,
    out_shape=_f32(NP, D),
)


def _k4_body(t2p_ref, t1_ref, hn_ref, dist_ref, m_ref, d2_ref, x1_ref,
             dis_ref, w2_ref, b1_ref, hd2_ref):
    hn = hn_ref[...]
    y = t2p_ref[0, :, :] + t2p_ref[1, :, :] + 2.0 * t1_ref[...] \
        - d2_ref[:, 0:1] * hn
    x2 = m_ref[:, 0:1] * jnp.maximum(dist_ref[:, 0:1] * (y + hn) + b1_ref[...],
                                     0.0)
    z = x1_ref[...] + x2
    hd2_ref[...] = dis_ref[:, 0:1] * jnp.dot(z, w2_ref[...],
                                             preferred_element_type=jnp.float32)


_k4 = pl.pallas_call(
    _k4_body,
    grid=(GRID,),
    in_specs=[_PARTSPEC, _ROWSPEC, _ROWSPEC, _S16SPEC, _ROWSPEC, _S16SPEC,
              _ROWSPEC, _S16SPEC, _WSPEC, _BSPEC],
    out_specs=_ROWSPEC,
    out_shape=_f32(NP, D),
)


def _k5_body(sp_ref, hd2_ref, dis_ref, b2_ref, w0n_ref, hdn_ref):
    dis = dis_ref[:, 0:1]
    o = dis * (sp_ref[0, :, :] + sp_ref[1, :, :] + hd2_ref[...]) + b2_ref[...]
    xn = jnp.where(o > 0, o, jnp.exp(o) - 1.0)
    hdn_ref[...] = dis * jnp.dot(xn, w0n_ref[...],
                                 preferred_element_type=jnp.float32)


_k5 = pl.pallas_call(
    _k5_body,
    grid=(GRID,),
    in_specs=[_PARTSPEC, _ROWSPEC, _S16SPEC, _BSPEC, _WSPEC],
    out_specs=_ROWSPEC,
    out_shape=_f32(NP, D),
)


def _k5f_body(sp_ref, hd2_ref, dis_ref, b2_ref, out_ref):
    dis = dis_ref[:, 0:1]
    o = dis * (sp_ref[0, :, :] + sp_ref[1, :, :] + hd2_ref[...]) + b2_ref[...]
    out_ref[...] = jnp.where(o > 0, o, float(D) * (jnp.exp(o) - 1.0))


_k5f = pl.pallas_call(
    _k5f_body,
    grid=(GRID,),
    in_specs=[_PARTSPEC, _ROWSPEC, _S16SPEC, _BSPEC],
    out_specs=_ROWSPEC,
    out_shape=_f32(NP, D),
)


def kernel(x, edge_index, edge_attr, params):
    row = edge_index[0].astype(jnp.int32)
    col = edge_index[1].astype(jnp.int32)
    nonself = row != col
    colb = jnp.where(nonself, col, TRASH)

    pad = E_PAD - NE
    rowp = jnp.concatenate([row, jnp.zeros((pad,), jnp.int32)])
    colp = jnp.concatenate([col, jnp.full((pad,), TRASH, jnp.int32)])
    colbp = jnp.concatenate([colb, jnp.full((pad,), TRASH, jnp.int32)])

    ekey = jnp.where(nonself, row * N + col, -1)
    rkey = col * N + row
    tagged = jnp.concatenate([ekey * 2, rkey * 2 + 1])
    rowpay = jnp.concatenate(
        [jnp.full((NE,), TRASH, jnp.int32), jnp.where(nonself, row, TRASH)])
    skey, srcrow = lax.sort_key_val(tagged, rowpay)
    key = skey >> 1
    is0i = ((skey & 1) == 0).astype(jnp.int32)
    newrun = jnp.concatenate(
        [jnp.ones((1,), jnp.bool_), key[1:] != key[:-1]])
    c0 = jnp.cumsum(is0i)
    seg_base = lax.cummax(jnp.where(newrun, c0 - is0i, -1))
    cntpos = (c0 - seg_base).astype(jnp.float32)
    vals = jnp.where(is0i == 1, 0.0, cntpos)
    trow = jnp.where(is0i == 1, TRASH, srcrow)
    se_pad = 2 * E_PAD
    val128s = jnp.broadcast_to(
        jnp.pad(vals, (0, se_pad - 2 * NE))[:, None], (se_pad, D))
    trow3 = jnp.pad(trow, (0, se_pad - 2 * NE),
                    constant_values=TRASH).reshape(2, NW, NCH, CHUNK)
    seidx = jnp.arange(se_pad, dtype=jnp.int32).reshape(2, NW, EPT)

    row_v = rowp.reshape(NW, EPT)
    row_v3 = rowp.reshape(NW, NCH, CHUNK)
    col_v3 = colp.reshape(NW, NCH, CHUNK)
    colb_v3 = colbp.reshape(NW, NCH, CHUNK)

    x_pad = jnp.pad(x, ((0, NP - N), (0, 0)))
    zeros_nd = jnp.zeros((NP, D), jnp.float32)
    ones_nd = jnp.ones((NP, D), jnp.float32)

    d2parts_a = _sc_edge_pass(val128s, seidx[0], trow3[0], zeros_nd)
    d2parts_b = _sc_edge_pass(val128s, seidx[1], trow3[1], zeros_nd)
    degparts = _sc_edge_pass(ones_nd, row_v, col_v3, zeros_nd)

    p1 = params['u1']
    hd, dis16, d2_16 = _k0(x_pad, d2parts_a, d2parts_b, degparts,
                           p1['down0_W'])

    ps = [params['u1'], params['u2'], params['u3'], params['u4']]
    for li, p in enumerate(ps):
        b0 = p['down0_b'].reshape(1, D)
        pw = p['pool_w'].reshape(1, D)
        b1 = p['down1_b'].reshape(1, D)
        b2 = p['up_b'].reshape(1, D)

        sparts = _sc_edge_pass(hd, row_v, col_v3, zeros_nd)
        x1, sc16, key16 = _k1(sparts, hd, dis16, b0, pw)
        m128 = _k2(key16)
        t1mparts = _sc_edge_pass(m128, row_v, colb_v3, zeros_nd)
        t1m = _k3b(t1mparts)
        t2mparts = _sc_edge_pass(t1m, row_v, colb_v3, zeros_nd)
        hn, dist16 = _k3(x1, sc16, m128, t1m, t2mparts, d2_16, p['down1_W'])
        t1parts = _sc_edge_pass(hn, row_v, colb_v3, zeros_nd)
        t1 = _k3b(t1parts)
        t2parts = _sc_edge_pass(t1, row_v, colb_v3, zeros_nd)
        hd2 = _k4(t2parts, t1, hn, dist16, m128, d2_16, x1, dis16,
                  p['up_W'], b1)
        s2parts = _sc_edge_pass(hd2, row_v, col_v3, zeros_nd)
        if li < 3:
            hd = _k5(s2parts, hd2, dis16, b2, ps[li + 1]['down0_W'])
        else:
            out = _k5f(s2parts, hd2, dis16, b2)

    return out[:N]
